# baseline (speedup 1.0000x reference)
import functools

import jax
import jax.numpy as jnp
from jax import lax
from jax.experimental import pallas as pl
from jax.experimental.pallas import tpu as pltpu
from jax.experimental.pallas import tpu_sc as plsc

N = 10000
E = 320000
D = 128

NC = 2
NS = 16
NW = NC * NS

CH = 32
GB = 8
GRP = GB * CH
EPAD = ((E + NW * GRP - 1) // (NW * GRP)) * (NW * GRP)
EPW = EPAD // NW
NGTOT = EPAD // GRP
NG0 = NGTOT // NS
NP = NG0 // 2
NPAD = 10240
RPT = NPAD // NS
PADIDX = N + 8

DGB = 8
DNG = EPW // (DGB * CH)

_mesh = plsc.VectorSubcoreMesh(core_axis_name="c", subcore_axis_name="s")


@functools.partial(
    pl.kernel,
    out_type=jax.ShapeDtypeStruct((NC, NPAD), jnp.float32),
    mesh=_mesh,
    scratch_types=[
        pltpu.VMEM((DNG, DGB, CH), jnp.int32),
        pltpu.VMEM((CH,), jnp.float32),
        pltpu.VMEM((RPT,), jnp.float32),
        pltpu.VMEM_SHARED((NPAD,), jnp.float32),
        pltpu.SemaphoreType.DMA,
    ],
)
def _sc_degree(dst_hbm, out_hbm, idx_v, ones_v, zeros_v, deg_sh, sem):
    c = lax.axis_index("c")
    s = lax.axis_index("s")
    wid = c * NS + s
    pltpu.sync_copy(dst_hbm.at[wid], idx_v)
    for i in range(CH // 16):
        ones_v[pl.ds(16 * i, 16)] = jnp.ones((16,), jnp.float32)
    for i in range(RPT // 16):
        zeros_v[pl.ds(16 * i, 16)] = jnp.zeros((16,), jnp.float32)
    pltpu.sync_copy(zeros_v, deg_sh.at[pl.ds(s * RPT, RPT)])
    plsc.subcore_barrier()

    def body(g, carry):
        cps = [
            pltpu.async_copy(ones_v, deg_sh.at[idx_v.at[g, b]], sem, add=True)
            for b in range(DGB)
        ]
        for cp in cps:
            cp.wait()
        return carry

    lax.fori_loop(0, DNG, body, 0)
    plsc.subcore_barrier()
    pltpu.sync_copy(deg_sh.at[pl.ds(s * RPT, RPT)], out_hbm.at[c, pl.ds(s * RPT, RPT)])


@functools.partial(
    pl.kernel,
    out_type=jax.ShapeDtypeStruct((NPAD, D), jnp.float32),
    mesh=_mesh,
    scratch_types=[
        pltpu.VMEM((2, 2, GB, CH), jnp.int32),
        pltpu.VMEM((GB, CH, D), jnp.float32),
        pltpu.VMEM_SHARED((NPAD, D), jnp.float32),
        pltpu.SemaphoreType.DMA,
        pltpu.SemaphoreType.DMA,
        pltpu.SemaphoreType.DMA,
    ],
)
def _sc_edge_sum(ei_hbm, g_hbm, out_hbm, idx_v, rows_v, acc_sh, semg, sems, semi):
    c = lax.axis_index("c")
    s = lax.axis_index("s")
    gbase = s * NG0

    @pl.when(c == 0)
    def _sc0_only():
        def zero_row(i, carry):
            for k in range(D // 16):
                rows_v[0, i, pl.ds(16 * k, 16)] = jnp.zeros((16,), jnp.float32)
            return carry

        lax.fori_loop(0, CH, zero_row, 0)
        for t in range(RPT // CH):
            pltpu.sync_copy(rows_v.at[0], acc_sh.at[pl.ds(s * RPT + t * CH, CH)])
        plsc.subcore_barrier()

        def process(slot):
            gcps = [
                pltpu.async_copy(g_hbm.at[idx_v.at[slot, 0, b]], rows_v.at[b], semg)
                for b in range(GB)
            ]
            scps = []
            for b in range(GB):
                gcps[b].wait()
                scps.append(
                    pltpu.async_copy(
                        rows_v.at[b], acc_sh.at[idx_v.at[slot, 1, b]], sems, add=True
                    )
                )
            for cp in scps:
                cp.wait()

        def wait_idx(slot):
            pltpu.make_async_copy(ei_hbm.at[:, 0], idx_v.at[slot], semi).wait()

        pltpu.sync_copy(ei_hbm.at[:, gbase], idx_v.at[0])
        pltpu.async_copy(ei_hbm.at[:, gbase + 1], idx_v.at[1], semi)

        def body(p, carry):
            process(0)
            wait_idx(1)

            @pl.when(p < NP - 1)
            def _():
                pltpu.async_copy(ei_hbm.at[:, gbase + 2 * p + 2], idx_v.at[0], semi)

            process(1)

            @pl.when(p < NP - 1)
            def _():
                wait_idx(0)
                pltpu.async_copy(ei_hbm.at[:, gbase + 2 * p + 3], idx_v.at[1], semi)

            return carry

        lax.fori_loop(0, NP, body, 0)
        plsc.subcore_barrier()
        pltpu.sync_copy(acc_sh.at[pl.ds(s * RPT, RPT)], out_hbm.at[pl.ds(s * RPT, RPT)])


BR = 1280


def _tc1_body(x_ref, w_ref, deg_ref, g_ref):
    dis = lax.rsqrt(deg_ref[0, :] + deg_ref[1, :] + 1.0)
    h = jnp.dot(x_ref[...], w_ref[...], preferred_element_type=jnp.float32)
    g_ref[...] = h * dis[:, None]


def _tc2_body(acc_ref, g1_ref, deg_ref, b_ref, w_ref, g2_ref):
    dis = lax.rsqrt(deg_ref[0, :] + deg_ref[1, :] + 1.0)
    tot = acc_ref[...] + g1_ref[...]
    h = jnp.maximum(tot * dis[:, None] + b_ref[...][None, :], 0.0)
    h2 = jnp.dot(h, w_ref[...], preferred_element_type=jnp.float32)
    g2_ref[...] = h2 * dis[:, None]


def _tc3_body(acc_ref, g2_ref, deg_ref, b_ref, out_ref):
    dis = lax.rsqrt(deg_ref[0, :] + deg_ref[1, :] + 1.0)
    tot = acc_ref[...] + g2_ref[...]
    out_ref[...] = jax.nn.sigmoid(tot * dis[:, None] + b_ref[...][None, :])


_GRID = NPAD // BR

_tc1 = pl.pallas_call(
    _tc1_body,
    grid=(_GRID,),
    in_specs=[
        pl.BlockSpec((BR, D), lambda i: (i, 0)),
        pl.BlockSpec((D, D), lambda i: (0, 0)),
        pl.BlockSpec((NC, BR), lambda i: (0, i)),
    ],
    out_specs=pl.BlockSpec((BR, D), lambda i: (i, 0)),
    out_shape=jax.ShapeDtypeStruct((NPAD, D), jnp.float32),
)

_tc2 = pl.pallas_call(
    _tc2_body,
    grid=(_GRID,),
    in_specs=[
        pl.BlockSpec((BR, D), lambda i: (i, 0)),
        pl.BlockSpec((BR, D), lambda i: (i, 0)),
        pl.BlockSpec((NC, BR), lambda i: (0, i)),
        pl.BlockSpec((D,), lambda i: (0,)),
        pl.BlockSpec((D, D), lambda i: (0, 0)),
    ],
    out_specs=pl.BlockSpec((BR, D), lambda i: (i, 0)),
    out_shape=jax.ShapeDtypeStruct((NPAD, D), jnp.float32),
)

_tc3 = pl.pallas_call(
    _tc3_body,
    grid=(_GRID,),
    in_specs=[
        pl.BlockSpec((BR, D), lambda i: (i, 0)),
        pl.BlockSpec((BR, D), lambda i: (i, 0)),
        pl.BlockSpec((NC, BR), lambda i: (0, i)),
        pl.BlockSpec((D,), lambda i: (0,)),
    ],
    out_specs=pl.BlockSpec((BR, D), lambda i: (i, 0)),
    out_shape=jax.ShapeDtypeStruct((NPAD, D), jnp.float32),
)


@jax.jit
def kernel(x, edge_index, W1, b1, W2, b2):
    padi = N + jnp.arange(EPAD - E, dtype=edge_index.dtype) % (NPAD - N)
    ei = jnp.concatenate([edge_index, jnp.stack([padi, padi])], axis=1)
    ei_g = ei.reshape(2, NGTOT, GB, CH)
    dst_g = ei[1].reshape(NW, DNG, DGB, CH)
    xp = jnp.pad(x, ((0, NPAD - N), (0, 0)))

    degp = _sc_degree(dst_g)
    g1 = _tc1(xp, W1, degp)
    acc1 = _sc_edge_sum(ei_g, g1)
    g2 = _tc2(acc1, g1, degp, b1, W2)
    acc2 = _sc_edge_sum(ei_g, g2)
    out = _tc3(acc2, g2, degp, b2)
    return out[:N]


# speedup vs baseline: 1.0259x; 1.0259x over previous
import functools

import jax
import jax.numpy as jnp
from jax import lax
from jax.experimental import pallas as pl
from jax.experimental.pallas import tpu as pltpu
from jax.experimental.pallas import tpu_sc as plsc

N = 10000
E = 320000
D = 128

NC = 2
NS = 16
NW = NC * NS

CH = 64
GB = 5
GRP = GB * CH
EPAD = ((E + NW * GRP - 1) // (NW * GRP)) * (NW * GRP)
EPW = EPAD // NW
NGTOT = EPAD // GRP
NG0 = NGTOT // NS
NP = NG0 // 2
NPAD = 10240
RPT = NPAD // NS

DGB = 8
DNG = EPW // (DGB * CH)

_mesh = plsc.VectorSubcoreMesh(core_axis_name="c", subcore_axis_name="s")


@functools.partial(
    pl.kernel,
    out_type=jax.ShapeDtypeStruct((NC, NPAD), jnp.float32),
    mesh=_mesh,
    scratch_types=[
        pltpu.VMEM((DNG, DGB, CH), jnp.int32),
        pltpu.VMEM((CH,), jnp.float32),
        pltpu.VMEM((RPT,), jnp.float32),
        pltpu.VMEM_SHARED((NPAD,), jnp.float32),
        pltpu.SemaphoreType.DMA,
    ],
)
def _sc_degree(dst_hbm, out_hbm, idx_v, ones_v, zeros_v, deg_sh, sem):
    c = lax.axis_index("c")
    s = lax.axis_index("s")
    wid = c * NS + s
    pltpu.sync_copy(dst_hbm.at[wid], idx_v)
    for i in range(CH // 16):
        ones_v[pl.ds(16 * i, 16)] = jnp.ones((16,), jnp.float32)
    for i in range(RPT // 16):
        zeros_v[pl.ds(16 * i, 16)] = jnp.zeros((16,), jnp.float32)
    pltpu.sync_copy(zeros_v, deg_sh.at[pl.ds(s * RPT, RPT)])
    plsc.subcore_barrier()

    def body(g, carry):
        cps = [
            pltpu.async_copy(ones_v, deg_sh.at[idx_v.at[g, b]], sem, add=True)
            for b in range(DGB)
        ]
        for cp in cps:
            cp.wait()
        return carry

    lax.fori_loop(0, DNG, body, 0)
    plsc.subcore_barrier()
    pltpu.sync_copy(deg_sh.at[pl.ds(s * RPT, RPT)], out_hbm.at[c, pl.ds(s * RPT, RPT)])


@functools.partial(
    pl.kernel,
    out_type=jax.ShapeDtypeStruct((NPAD, D), jnp.float32),
    mesh=_mesh,
    scratch_types=[
        pltpu.VMEM((2, 2, GB, CH), jnp.int32),
        pltpu.VMEM((GB, CH, D), jnp.float32),
        pltpu.VMEM_SHARED((NPAD, D), jnp.float32),
        pltpu.SemaphoreType.DMA,
        pltpu.SemaphoreType.DMA,
        pltpu.SemaphoreType.DMA,
    ],
)
def _sc_edge_sum(ei_hbm, g_hbm, out_hbm, idx_v, rows_v, acc_sh, semg, sems, semi):
    c = lax.axis_index("c")
    s = lax.axis_index("s")
    gbase = s * NG0

    @pl.when(c == 0)
    def _sc0_only():
        def zero_row(i, carry):
            for k in range(D // 16):
                rows_v[0, i, pl.ds(16 * k, 16)] = jnp.zeros((16,), jnp.float32)
            return carry

        lax.fori_loop(0, CH, zero_row, 0)
        for t in range(RPT // CH):
            pltpu.sync_copy(rows_v.at[0], acc_sh.at[pl.ds(s * RPT + t * CH, CH)])
        plsc.subcore_barrier()

        def process(slot):
            gcps = [
                pltpu.async_copy(g_hbm.at[idx_v.at[slot, 0, b]], rows_v.at[b], semg)
                for b in range(GB)
            ]
            scps = []
            for b in range(GB):
                gcps[b].wait()
                scps.append(
                    pltpu.async_copy(
                        rows_v.at[b], acc_sh.at[idx_v.at[slot, 1, b]], sems, add=True
                    )
                )
            for cp in scps:
                cp.wait()

        def wait_idx(slot):
            pltpu.make_async_copy(ei_hbm.at[:, 0], idx_v.at[slot], semi).wait()

        pltpu.sync_copy(ei_hbm.at[:, gbase], idx_v.at[0])
        pltpu.async_copy(ei_hbm.at[:, gbase + 1], idx_v.at[1], semi)

        def body(p, carry):
            process(0)
            wait_idx(1)

            @pl.when(p < NP - 1)
            def _():
                pltpu.async_copy(ei_hbm.at[:, gbase + 2 * p + 2], idx_v.at[0], semi)

            process(1)

            @pl.when(p < NP - 1)
            def _():
                wait_idx(0)
                pltpu.async_copy(ei_hbm.at[:, gbase + 2 * p + 3], idx_v.at[1], semi)

            return carry

        lax.fori_loop(0, NP, body, 0)
        plsc.subcore_barrier()
        pltpu.sync_copy(acc_sh.at[pl.ds(s * RPT, RPT)], out_hbm.at[pl.ds(s * RPT, RPT)])


BR = 1280


def _tc1_body(x_ref, w_ref, deg_ref, g_ref):
    dis = lax.rsqrt(deg_ref[0, :] + deg_ref[1, :] + 1.0)
    h = jnp.dot(x_ref[...], w_ref[...], preferred_element_type=jnp.float32)
    g_ref[...] = h * dis[:, None]


def _tc2_body(acc_ref, g1_ref, deg_ref, b_ref, w_ref, g2_ref):
    dis = lax.rsqrt(deg_ref[0, :] + deg_ref[1, :] + 1.0)
    tot = acc_ref[...] + g1_ref[...]
    h = jnp.maximum(tot * dis[:, None] + b_ref[...][None, :], 0.0)
    h2 = jnp.dot(h, w_ref[...], preferred_element_type=jnp.float32)
    g2_ref[...] = h2 * dis[:, None]


def _tc3_body(acc_ref, g2_ref, deg_ref, b_ref, out_ref):
    dis = lax.rsqrt(deg_ref[0, :] + deg_ref[1, :] + 1.0)
    tot = acc_ref[...] + g2_ref[...]
    out_ref[...] = jax.nn.sigmoid(tot * dis[:, None] + b_ref[...][None, :])


_GRID = NPAD // BR

_tc1 = pl.pallas_call(
    _tc1_body,
    grid=(_GRID,),
    in_specs=[
        pl.BlockSpec((BR, D), lambda i: (i, 0)),
        pl.BlockSpec((D, D), lambda i: (0, 0)),
        pl.BlockSpec((NC, BR), lambda i: (0, i)),
    ],
    out_specs=pl.BlockSpec((BR, D), lambda i: (i, 0)),
    out_shape=jax.ShapeDtypeStruct((NPAD, D), jnp.float32),
)

_tc2 = pl.pallas_call(
    _tc2_body,
    grid=(_GRID,),
    in_specs=[
        pl.BlockSpec((BR, D), lambda i: (i, 0)),
        pl.BlockSpec((BR, D), lambda i: (i, 0)),
        pl.BlockSpec((NC, BR), lambda i: (0, i)),
        pl.BlockSpec((D,), lambda i: (0,)),
        pl.BlockSpec((D, D), lambda i: (0, 0)),
    ],
    out_specs=pl.BlockSpec((BR, D), lambda i: (i, 0)),
    out_shape=jax.ShapeDtypeStruct((NPAD, D), jnp.float32),
)

_tc3 = pl.pallas_call(
    _tc3_body,
    grid=(_GRID,),
    in_specs=[
        pl.BlockSpec((BR, D), lambda i: (i, 0)),
        pl.BlockSpec((BR, D), lambda i: (i, 0)),
        pl.BlockSpec((NC, BR), lambda i: (0, i)),
        pl.BlockSpec((D,), lambda i: (0,)),
    ],
    out_specs=pl.BlockSpec((BR, D), lambda i: (i, 0)),
    out_shape=jax.ShapeDtypeStruct((NPAD, D), jnp.float32),
)


@jax.jit
def kernel(x, edge_index, W1, b1, W2, b2):
    padi = N + jnp.arange(EPAD - E, dtype=edge_index.dtype) % (NPAD - N)
    ei = jnp.concatenate([edge_index, jnp.stack([padi, padi])], axis=1)
    ei_g = ei.reshape(2, NGTOT, GB, CH)
    dst_g = ei[1].reshape(NW, DNG, DGB, CH)
    xp = jnp.pad(x, ((0, NPAD - N), (0, 0)))

    degp = _sc_degree(dst_g)
    g1 = _tc1(xp, W1, degp)
    acc1 = _sc_edge_sum(ei_g, g1)
    g2 = _tc2(acc1, g1, degp, b1, W2)
    acc2 = _sc_edge_sum(ei_g, g2)
    out = _tc3(acc2, g2, degp, b2)
    return out[:N]
